# baseline (speedup 1.0000x reference)
import jax
import jax.numpy as jnp
from jax import lax
from jax.experimental import pallas as pl
from jax.experimental.pallas import tpu as pltpu
from jax.experimental.pallas import tpu_sc as plsc

_HIDDEN = 1024
_ROWS = 4096
_NC, _NS, _LANES = 2, 16, 16
_NW = _NC * _NS
_RPW = _ROWS // _NW
_CHUNK = 32


def _body(feat_hbm, idx_hbm, table_hbm, out_hbm, idx_v, feat_v, rows_v,
          sem_f, sem_g):
    wid = lax.axis_index("s") * _NC + lax.axis_index("c")
    base = wid * _RPW
    pltpu.sync_copy(idx_hbm.at[pl.ds(base, _RPW)], idx_v)
    for c in range(_RPW // _CHUNK):
        row0 = base + c * _CHUNK
        cp_f = pltpu.async_copy(feat_hbm.at[pl.ds(row0, _CHUNK)], feat_v,
                                sem_f)
        cp_g = pltpu.async_copy(table_hbm.at[idx_v.at[pl.ds(c * _CHUNK,
                                                            _CHUNK)]],
                                rows_v, sem_g)
        cp_f.wait()
        cp_g.wait()

        def add_row(r, carry):
            for j in range(_HIDDEN // _LANES):
                sl = pl.ds(j * _LANES, _LANES)
                feat_v[r, sl] = feat_v[r, sl] + rows_v[r, sl]
            return carry

        lax.fori_loop(0, _CHUNK, add_row, 0)
        pltpu.sync_copy(feat_v, out_hbm.at[pl.ds(row0, _CHUNK)])


def kernel(features, frame_positions, temporal_pos_embedding_weight):
    b, f, h = features.shape
    feat2 = features.reshape(b * f, h)
    idx = frame_positions.reshape(b * f)
    mesh = plsc.VectorSubcoreMesh(core_axis_name="c", subcore_axis_name="s")
    out = pl.kernel(
        _body,
        out_type=jax.ShapeDtypeStruct((b * f, h), jnp.float32),
        mesh=mesh,
        scratch_types=[
            pltpu.VMEM((_RPW,), jnp.int32),
            pltpu.VMEM((_CHUNK, _HIDDEN), jnp.float32),
            pltpu.VMEM((_CHUNK, _HIDDEN), jnp.float32),
            pltpu.SemaphoreType.DMA,
            pltpu.SemaphoreType.DMA,
        ],
    )(feat2, idx, temporal_pos_embedding_weight)
    return out.reshape(b, f, h)
